# baseline (speedup 1.0000x reference)
import functools

import jax
import jax.numpy as jnp
from jax import lax
from jax.experimental import pallas as pl
from jax.experimental.pallas import tpu as pltpu
from jax.experimental.pallas import tpu_sc as plsc

B, N, D = 16, 2048, 2048
ROWS = B * N
SB = 2
TCK = 4
TR = N // TCK

NC, NS, L = 2, 16, 16
NW = NC * NS
WPB = NW // SB
SC_ROWS = SB * N
RPW = SC_ROWS // NW
CH = 16
NCH = RPW // CH
RG = 8
NJ = D // L

H1, H1P = 200, 208
H2, H2P = 100, 112


def _lane_gather(a, idx):
    return lax.gather(
        a, idx[:, None],
        lax.GatherDimensionNumbers(
            offset_dims=(), collapsed_slice_dims=(0,), start_index_map=(0,)),
        slice_sizes=(1,),
        mode=lax.GatherScatterMode.PROMISE_IN_BOUNDS)


def _bfly(a, op):
    for sft in (8, 4, 2, 1):
        lane = lax.broadcasted_iota(jnp.int32, (L,), 0)
        a = op(a, _lane_gather(a, lane ^ sft))
    return a


def _sig(z):
    return 1.0 / (1.0 + jnp.exp(-z))


_mesh = plsc.VectorSubcoreMesh(
    core_axis_name="c", subcore_axis_name="s", num_cores=NC, num_subcores=NS)


@functools.partial(
    pl.kernel,
    out_type=jax.ShapeDtypeStruct((SB, L), jnp.float32),
    mesh=_mesh,
    scratch_types=[
        pltpu.VMEM((D,), jnp.float32),
        pltpu.VMEM((2, CH, D), jnp.float32),
        pltpu.VMEM((L,), jnp.float32),
        pltpu.VMEM((WPB, L), jnp.float32),
        pltpu.VMEM((4, H1P), jnp.float32),
        pltpu.VMEM((H1P,), jnp.float32),
        pltpu.VMEM((H1P,), jnp.float32),
        pltpu.VMEM((H1P * H2P,), jnp.float32),
        pltpu.VMEM((H2P,), jnp.float32),
        pltpu.VMEM((H2P,), jnp.float32),
        pltpu.VMEM((L,), jnp.float32),
        pltpu.VMEM((L,), jnp.float32),
        pltpu.VMEM_SHARED((NS, L), jnp.float32),
        pltpu.SemaphoreType.DMA,
        pltpu.SemaphoreType.DMA,
    ],
)
def _sc_part(x_hbm, w_hbm, w1tp_hbm, b1p_hbm, w2tpf_hbm, b2p_hbm, w3p_hbm,
             b3p_hbm, out_hbm, w_v, bufs, cand_v, cand4_v, w1tp_v, b1p_v,
             h1_v, w2tpf_v, b2p_v, w3p_v, b3p_v, out_v, board, sem0, sem1):
    c_idx = lax.axis_index("c")
    s_idx = lax.axis_index("s")
    wid = c_idx * NS + s_idx
    base_row = wid * RPW
    is_leader = (wid % WPB) == 0
    bag = wid // WPB
    sems = (sem0, sem1)
    lane = lax.broadcasted_iota(jnp.int32, (L,), 0)

    pltpu.sync_copy(w_hbm, w_v)

    @pl.when(is_leader)
    def _load_head():
        pltpu.sync_copy(w1tp_hbm, w1tp_v)
        pltpu.sync_copy(b1p_hbm, b1p_v)
        pltpu.sync_copy(w2tpf_hbm, w2tpf_v)
        pltpu.sync_copy(b2p_hbm, b2p_v)
        pltpu.sync_copy(w3p_hbm, w3p_v)
        pltpu.sync_copy(b3p_hbm, b3p_v)

    def chunk_src(c):
        return x_hbm.at[pl.ds(base_row + c * CH, CH), :]

    for b in (0, 1):
        pltpu.async_copy(chunk_src(b), bufs.at[b], sems[b])

    inf = jnp.float32(jnp.inf)
    init = (jnp.full((L,), inf, jnp.float32),
            jnp.full((L,), inf, jnp.float32),
            jnp.full((L,), -inf, jnp.float32),
            jnp.full((L,), -inf, jnp.float32))

    def outer(g, carry):
        vmin1, vmin2, vmax1, vmax2 = carry
        for b in (0, 1):
            c = 2 * g + b
            pltpu.make_async_copy(chunk_src(c), bufs.at[b], sems[b]).wait()

            sv = jnp.zeros((L,), jnp.float32)
            for rg in range(CH // RG):
                def jbody(j, accs, _rg=rg, _b=b):
                    wv = w_v[pl.ds(j * L, L)]
                    return tuple(
                        accs[r] + bufs[_b, _rg * RG + r, pl.ds(j * L, L)] * wv
                        for r in range(RG))
                accs = lax.fori_loop(
                    0, NJ, jbody,
                    tuple(jnp.zeros((L,), jnp.float32) for _ in range(RG)),
                    unroll=8)
                for r in range(RG):
                    a = _bfly(accs[r], jnp.add)
                    sv = jnp.where(lane == (rg * RG + r), a, sv)

            vmin2 = jnp.minimum(vmin2, jnp.maximum(vmin1, sv))
            vmin1 = jnp.minimum(vmin1, sv)
            vmax2 = jnp.maximum(vmax2, jnp.minimum(vmax1, sv))
            vmax1 = jnp.maximum(vmax1, sv)

            @pl.when(c + 2 < NCH)
            def _():
                pltpu.async_copy(chunk_src(c + 2), bufs.at[b], sems[b])
        return vmin1, vmin2, vmax1, vmax2

    vmin1, vmin2, vmax1, vmax2 = lax.fori_loop(0, NCH // 2, outer, init)

    def two_smallest(v1, v2):
        m1 = _bfly(v1, jnp.minimum)
        idx1 = _bfly(jnp.where(v1 == m1, lane, L), jnp.minimum)
        vrest = jnp.where(lane == idx1, v2, v1)
        m2 = _bfly(vrest, jnp.minimum)
        return m1, m2

    m1, m2 = two_smallest(vmin1, vmin2)
    nM1, nM2 = two_smallest(-vmax1, -vmax2)
    M1, M2 = -nM1, -nM2

    cand = jnp.where(lane == 0, m1,
                     jnp.where(lane == 1, m2,
                               jnp.where(lane == 2, M1,
                                         jnp.where(lane == 3, M2,
                                                   jnp.zeros((L,),
                                                             jnp.float32)))))
    cand_v[...] = cand
    pltpu.sync_copy(cand_v, board.at[s_idx])
    plsc.subcore_barrier()

    @pl.when(is_leader)
    def _finish():
        pltpu.sync_copy(board.at[pl.ds(s_idx, WPB)], cand4_v)

        def merge2(a1, a2, b1, b2):
            return (jnp.minimum(a1, b1),
                    jnp.minimum(jnp.maximum(a1, b1), jnp.minimum(a2, b2)))

        row0 = cand4_v[0, pl.ds(0, L)]
        fmin1, fmin2 = row0[0], row0[1]
        fmax1, fmax2 = -row0[2], -row0[3]
        for r in range(1, WPB):
            rw = cand4_v[r, pl.ds(0, L)]
            fmin1, fmin2 = merge2(fmin1, fmin2, rw[0], rw[1])
            fmax1, fmax2 = merge2(fmax1, fmax2, -rw[2], -rw[3])
        f = (fmin1, fmin2, -fmax1, -fmax2)

        for t in range(H1P // L):
            sl = pl.ds(t * L, L)
            h = b1p_v[sl]
            for k in range(4):
                h = h + f[k] * w1tp_v[k, sl]
            h1_v[sl] = _sig(h)

        def l2body(t1, accs):
            h1s = h1_v[pl.ds(t1 * L, L)]
            for kk in range(L):
                hk = h1s[kk]
                base = t1 * (L * H2P) + kk * H2P
                accs = tuple(
                    accs[t] + hk * w2tpf_v[pl.ds(base + t * L, L)]
                    for t in range(H2P // L))
            return accs
        accs = lax.fori_loop(
            0, H1P // L, l2body,
            tuple(jnp.zeros((L,), jnp.float32) for _ in range(H2P // L)))

        a3 = jnp.zeros((L,), jnp.float32)
        for t in range(H2P // L):
            sl = pl.ds(t * L, L)
            h2t = _sig(accs[t] + b2p_v[sl])
            a3 = a3 + h2t * w3p_v[sl]
        z = _bfly(a3, jnp.add) + b3p_v[...]
        out_v[...] = _sig(z)
        pltpu.sync_copy(out_v, out_hbm.at[bag])


def _tc_body(x_ref, w_ref, w1t_ref, b1_ref, w2t_ref, b2_ref, w3t_ref, b3_ref,
             o_ref):
    s = jax.lax.dot_general(
        x_ref[...], w_ref[...],
        dimension_numbers=(((1,), (0,)), ((), ())),
        preferred_element_type=jnp.float32,
    )
    iota = jax.lax.broadcasted_iota(jnp.int32, (N, 1), 0)

    max1 = jnp.max(s)
    idx_max = jnp.min(jnp.where(s == max1, iota, N))
    max2 = jnp.max(jnp.where(iota == idx_max, -jnp.inf, s))

    min1 = jnp.min(s)
    idx_min = jnp.min(jnp.where(s == min1, iota, N))
    min2 = jnp.min(jnp.where(iota == idx_min, jnp.inf, s))

    h = (b1_ref[...]
         + min1 * w1t_ref[0:1, :]
         + min2 * w1t_ref[1:2, :]
         + max1 * w1t_ref[2:3, :]
         + max2 * w1t_ref[3:4, :])
    h = jax.nn.sigmoid(h)

    h2 = jax.nn.sigmoid(
        jax.lax.dot_general(h, w2t_ref[...],
                            dimension_numbers=(((1,), (0,)), ((), ())),
                            preferred_element_type=jnp.float32)
        + b2_ref[...])

    i = pl.program_id(0)
    o_ref[pl.ds(i, 1), :] = jax.nn.sigmoid(
        jax.lax.dot_general(h2, w3t_ref[...],
                            dimension_numbers=(((1,), (0,)), ((), ())),
                            preferred_element_type=jnp.float32)
        + b3_ref[...])


def _tc_kernel(xf, wt, w1t, b1r, w2t, b2r, w3t, b3r):
    const = lambda i: (0, 0)
    return pl.pallas_call(
        _tc_body,
        grid=(B - SB,),
        in_specs=[
            pl.BlockSpec((N, D), lambda i: (SB + i, 0)),
            pl.BlockSpec((D, 1), const),
            pl.BlockSpec((4, 200), const),
            pl.BlockSpec((1, 200), const),
            pl.BlockSpec((200, 100), const),
            pl.BlockSpec((1, 100), const),
            pl.BlockSpec((100, 1), const),
            pl.BlockSpec((1, 1), const),
        ],
        out_specs=pl.BlockSpec((B - SB, 1), const),
        out_shape=jax.ShapeDtypeStruct((B - SB, 1), jnp.float32),
    )(xf, wt, w1t, b1r, w2t, b2r, w3t, b3r)


@jax.jit
def kernel(x, W_embed, W1, b1, W2, b2, W3, b3):
    xf = x.reshape(ROWS, D)
    wt = W_embed.reshape(D, 1)
    w1t = W1.T
    b1r = b1.reshape(1, 200)
    w2t = W2.T
    b2r = b2.reshape(1, 100)
    w3t = W3.T
    b3r = b3.reshape(1, 1)
    const = lambda i: (0, 0)

    w1tp = jnp.pad(W1.T, ((0, 0), (0, H1P - H1)))
    b1p = jnp.pad(b1, (0, H1P - H1))
    w2tpf = jnp.pad(W2.T, ((0, H1P - H1), (0, H2P - H2))).reshape(-1)
    b2p = jnp.pad(b2, (0, H2P - H2))
    w3p = jnp.pad(W3.reshape(-1), (0, H2P - H2))
    b3p = jnp.broadcast_to(b3, (L,))

    out_tc = _tc_kernel(xf, wt, w1t, b1r, w2t, b2r, w3t, b3r)

    out_sc = _sc_part(xf, W_embed.reshape(D), w1tp, b1p, w2tpf, b2p, w3p, b3p)

    return jnp.concatenate([out_sc[:, 0], out_tc.reshape(-1)], axis=0)


# speedup vs baseline: 1.0279x; 1.0279x over previous
import functools

import jax
import jax.numpy as jnp
from jax import lax
from jax.experimental import pallas as pl
from jax.experimental.pallas import tpu as pltpu
from jax.experimental.pallas import tpu_sc as plsc

B, N, D = 16, 2048, 2048
ROWS = B * N
SB = 4

NC, NS, L = 2, 16, 16
NW = NC * NS
SC_ROWS = SB * N
RPW = SC_ROWS // NW
CH = 16
NCH = RPW // CH
RG = 8
NJ = D // L


def _lane_gather(a, idx):
    return lax.gather(
        a, idx[:, None],
        lax.GatherDimensionNumbers(
            offset_dims=(), collapsed_slice_dims=(0,), start_index_map=(0,)),
        slice_sizes=(1,),
        mode=lax.GatherScatterMode.PROMISE_IN_BOUNDS)


_mesh = plsc.VectorSubcoreMesh(
    core_axis_name="c", subcore_axis_name="s", num_cores=NC, num_subcores=NS)


@functools.partial(
    pl.kernel,
    out_type=jax.ShapeDtypeStruct((SC_ROWS,), jnp.float32),
    mesh=_mesh,
    scratch_types=[
        pltpu.VMEM((D,), jnp.float32),
        pltpu.VMEM((2, CH, D), jnp.float32),
        pltpu.VMEM((RPW,), jnp.float32),
        pltpu.SemaphoreType.DMA,
        pltpu.SemaphoreType.DMA,
    ],
)
def _sc_scores(x_hbm, w_hbm, out_hbm, w_v, bufs, sc_v, sem0, sem1):
    wid = lax.axis_index("s") * NC + lax.axis_index("c")
    base_row = wid * RPW
    sems = (sem0, sem1)
    lane = lax.broadcasted_iota(jnp.int32, (L,), 0)

    pltpu.sync_copy(w_hbm, w_v)

    def chunk_src(c):
        return x_hbm.at[pl.ds(base_row + c * CH, CH), :]

    for b in (0, 1):
        pltpu.async_copy(chunk_src(b), bufs.at[b], sems[b])

    def outer(g, _):
        for b in (0, 1):
            c = 2 * g + b
            pltpu.make_async_copy(chunk_src(c), bufs.at[b], sems[b]).wait()

            sv = jnp.zeros((L,), jnp.float32)
            for rg in range(CH // RG):
                def jbody(j, accs, _rg=rg, _b=b):
                    wv = w_v[pl.ds(j * L, L)]
                    return tuple(
                        accs[r] + bufs[_b, _rg * RG + r, pl.ds(j * L, L)] * wv
                        for r in range(RG))
                accs = lax.fori_loop(
                    0, NJ, jbody,
                    tuple(jnp.zeros((L,), jnp.float32) for _ in range(RG)),
                    unroll=8)
                for r in range(RG):
                    a = accs[r]
                    for sft in (8, 4, 2, 1):
                        a = a + _lane_gather(a, lane ^ sft)
                    sv = jnp.where(lane == (rg * RG + r), a, sv)
            sc_v[pl.ds(c * CH, CH)] = sv

            @pl.when(c + 2 < NCH)
            def _():
                pltpu.async_copy(chunk_src(c + 2), bufs.at[b], sems[b])
        return _

    lax.fori_loop(0, NCH // 2, outer, None)
    pltpu.sync_copy(sc_v, out_hbm.at[pl.ds(base_row, RPW)])


def _tc_body(x_ref, w_ref, w1t_ref, b1_ref, w2t_ref, b2_ref, w3t_ref, b3_ref,
             o_ref):
    s = jax.lax.dot_general(
        x_ref[...], w_ref[...],
        dimension_numbers=(((1,), (0,)), ((), ())),
        preferred_element_type=jnp.float32,
    )
    iota = jax.lax.broadcasted_iota(jnp.int32, (N, 1), 0)

    max1 = jnp.max(s)
    idx_max = jnp.min(jnp.where(s == max1, iota, N))
    max2 = jnp.max(jnp.where(iota == idx_max, -jnp.inf, s))

    min1 = jnp.min(s)
    idx_min = jnp.min(jnp.where(s == min1, iota, N))
    min2 = jnp.min(jnp.where(iota == idx_min, jnp.inf, s))

    h = (b1_ref[...]
         + min1 * w1t_ref[0:1, :]
         + min2 * w1t_ref[1:2, :]
         + max1 * w1t_ref[2:3, :]
         + max2 * w1t_ref[3:4, :])
    h = jax.nn.sigmoid(h)

    h2 = jax.nn.sigmoid(
        jax.lax.dot_general(h, w2t_ref[...],
                            dimension_numbers=(((1,), (0,)), ((), ())),
                            preferred_element_type=jnp.float32)
        + b2_ref[...])

    i = pl.program_id(0)
    o_ref[pl.ds(i, 1), :] = jax.nn.sigmoid(
        jax.lax.dot_general(h2, w3t_ref[...],
                            dimension_numbers=(((1,), (0,)), ((), ())),
                            preferred_element_type=jnp.float32)
        + b3_ref[...])


def _head_body(s_ref, w1t_ref, b1_ref, w2t_ref, b2_ref, w3t_ref, b3_ref,
               o_ref):
    s = s_ref[...]
    iota = jax.lax.broadcasted_iota(jnp.int32, (SB, N), 1)

    max1 = jnp.max(s, axis=1, keepdims=True)
    idx_max = jnp.min(jnp.where(s == max1, iota, N), axis=1, keepdims=True)
    max2 = jnp.max(jnp.where(iota == idx_max, -jnp.inf, s), axis=1,
                   keepdims=True)

    min1 = jnp.min(s, axis=1, keepdims=True)
    idx_min = jnp.min(jnp.where(s == min1, iota, N), axis=1, keepdims=True)
    min2 = jnp.min(jnp.where(iota == idx_min, jnp.inf, s), axis=1,
                   keepdims=True)

    f = jnp.concatenate([min1, min2, max1, max2], axis=1)

    h = b1_ref[...]
    w1t = w1t_ref[...]
    for k in range(4):
        h = h + f[:, k:k + 1] * w1t[k:k + 1, :]
    h = jax.nn.sigmoid(h)

    h2 = jax.nn.sigmoid(
        jax.lax.dot_general(h, w2t_ref[...],
                            dimension_numbers=(((1,), (0,)), ((), ())),
                            preferred_element_type=jnp.float32)
        + b2_ref[...])

    o_ref[...] = jax.nn.sigmoid(
        jax.lax.dot_general(h2, w3t_ref[...],
                            dimension_numbers=(((1,), (0,)), ((), ())),
                            preferred_element_type=jnp.float32)
        + b3_ref[...])


@jax.jit
def kernel(x, W_embed, W1, b1, W2, b2, W3, b3):
    xf = x.reshape(ROWS, D)
    wt = W_embed.reshape(D, 1)
    w1t = W1.T
    b1r = b1.reshape(1, 200)
    w2t = W2.T
    b2r = b2.reshape(1, 100)
    w3t = W3.T
    b3r = b3.reshape(1, 1)
    const = lambda i: (0, 0)

    sc_scores = _sc_scores(xf, W_embed.reshape(D))

    out_tc = pl.pallas_call(
        _tc_body,
        grid=(B - SB,),
        in_specs=[
            pl.BlockSpec((N, D), lambda i: (SB + i, 0)),
            pl.BlockSpec((D, 1), const),
            pl.BlockSpec((4, 200), const),
            pl.BlockSpec((1, 200), const),
            pl.BlockSpec((200, 100), const),
            pl.BlockSpec((1, 100), const),
            pl.BlockSpec((100, 1), const),
            pl.BlockSpec((1, 1), const),
        ],
        out_specs=pl.BlockSpec((B - SB, 1), const),
        out_shape=jax.ShapeDtypeStruct((B - SB, 1), jnp.float32),
    )(xf, wt, w1t, b1r, w2t, b2r, w3t, b3r)

    out_sc = pl.pallas_call(
        _head_body,
        out_shape=jax.ShapeDtypeStruct((SB, 1), jnp.float32),
    )(sc_scores.reshape(SB, N), w1t, b1r, w2t, b2r, w3t, b3r)

    return jnp.concatenate([out_sc, out_tc], axis=0).reshape(-1)
